# Initial kernel scaffold; baseline (speedup 1.0000x reference)
import jax
import jax.numpy as jnp
from jax.experimental import pallas as pl


def kernel(nodes, edges, W_in, b_in, W_gcn, b_gcn, W1, b1, W2, b2):
    raise NotImplementedError("write your pallas kernel here")


# speedup vs baseline: 4.3675x; 4.3675x over previous
import functools

import jax
import jax.numpy as jnp
from jax import lax
from jax.experimental import pallas as pl
from jax.experimental.pallas import tpu as pltpu
from jax.experimental.pallas import tpu_sc as plsc

N_NODES = 10000
N_PAD = 10240
D_FEAT = 128
D_HID = 512
D_MLP = 256
N_CLASS = 5
N_EDGES = 320000
NC = 2
NS = 16
KP = D_HID // 128
EDGE_CHUNK = 80
CHUNKS = N_EDGES // (NC * NS) // EDGE_CHUNK
VECS = CHUNKS * EDGE_CHUNK // 16
ROWS_PER_TILE = N_PAD // NS


def _sc_scatter_body(h0v_hbm, edges_hbm, out_hbm, idx_src, idx_dst, rows,
                     acc, sem):
    c = lax.axis_index("c")
    s = lax.axis_index("s")
    wid = s * NC + c

    pltpu.sync_copy(edges_hbm.at[0, wid], idx_src)
    pltpu.sync_copy(edges_hbm.at[1, wid], idx_dst)

    VPC = EDGE_CHUNK // 16

    def _scale(i, carry):
        r = i // VPC
        col = (i % VPC) * 16
        idx_src[r, pl.ds(col, 16)] = idx_src[r, pl.ds(col, 16)] * 4
        return carry

    lax.fori_loop(0, VECS, _scale, 0)

    for k in range(KP):
        if k > 0:
            def _bump(i, carry):
                r = i // VPC
                col = (i % VPC) * 16
                idx_src[r, pl.ds(col, 16)] = idx_src[r, pl.ds(col, 16)] + 1
                return carry

            lax.fori_loop(0, VECS, _bump, 0)

        def _zb(i, carry):
            r = i // 8
            col = (i % 8) * 16
            rows[r, pl.ds(col, 16)] = jnp.zeros((16,), jnp.float32)
            return carry

        lax.fori_loop(0, EDGE_CHUNK * 8, _zb, 0)

        def _zacc(i, carry):
            pltpu.sync_copy(
                rows,
                acc.at[pl.ds(s * ROWS_PER_TILE + i * EDGE_CHUNK, EDGE_CHUNK)])
            return carry

        lax.fori_loop(0, ROWS_PER_TILE // EDGE_CHUNK, _zacc, 0)

        plsc.subcore_barrier()

        def _chunk(j, carry):
            pltpu.async_copy(h0v_hbm.at[idx_src.at[j]], rows, sem).wait()
            pltpu.sync_copy(rows, acc.at[idx_dst.at[j]], add=True)
            return carry

        lax.fori_loop(0, CHUNKS, _chunk, 0)

        plsc.subcore_barrier()

        r0 = s * ROWS_PER_TILE
        pltpu.sync_copy(acc.at[pl.ds(r0, ROWS_PER_TILE)],
                        out_hbm.at[k, c, pl.ds(r0, ROWS_PER_TILE)])


_sc_scatter = functools.partial(
    pl.kernel,
    out_type=jax.ShapeDtypeStruct((KP, NC, N_PAD, 128), jnp.float32),
    mesh=plsc.VectorSubcoreMesh(core_axis_name="c", subcore_axis_name="s"),
    scratch_types=[
        pltpu.VMEM((CHUNKS, EDGE_CHUNK), jnp.int32),
        pltpu.VMEM((CHUNKS, EDGE_CHUNK), jnp.int32),
        pltpu.VMEM((EDGE_CHUNK, 128), jnp.float32),
        pltpu.VMEM_SHARED((N_PAD, 128), jnp.float32),
        pltpu.SemaphoreType.DMA,
    ],
)(_sc_scatter_body)


def _h0_body(nodes_ref, w_in_ref, b_in_ref, out_ref):
    out_ref[...] = jnp.dot(nodes_ref[...], w_in_ref[...],
                           preferred_element_type=jnp.float32) + b_in_ref[...]


_tc_h0 = pl.pallas_call(
    _h0_body,
    out_shape=jax.ShapeDtypeStruct((N_NODES, D_HID), jnp.float32),
)


ROW_BLK = 2000
N_BLKS = N_NODES // ROW_BLK


def _tc_body(h0_ref, part_ref, w_gcn_ref, b_gcn_ref,
             w1_ref, b1_ref, w2_ref, b2_ref, out_ref):
    i = pl.program_id(0)

    h0 = h0_ref[...]
    agg = jnp.concatenate(
        [part_ref[k, 0] + part_ref[k, 1] for k in range(KP)], axis=1)
    feature = jnp.maximum(
        jnp.dot(agg, w_gcn_ref[...], preferred_element_type=jnp.float32)
        + b_gcn_ref[...], 0.0) + h0
    x = jnp.maximum(
        jnp.dot(feature, w1_ref[...], preferred_element_type=jnp.float32)
        + b1_ref[...], 0.0)
    logits = jnp.dot(x, w2_ref[...],
                     preferred_element_type=jnp.float32) + b2_ref[...]
    out_ref[pl.ds(i * ROW_BLK, ROW_BLK), :] = logits

    @pl.when(i == N_BLKS - 1)
    def _():
        lg = out_ref[...]
        m = jnp.max(lg, axis=0, keepdims=True)
        e = jnp.exp(lg - m)
        out_ref[...] = e / jnp.sum(e, axis=0, keepdims=True)


def _full(shape):
    return pl.BlockSpec(shape, lambda i: (0,) * len(shape))


_tc_dense = pl.pallas_call(
    _tc_body,
    grid=(N_BLKS,),
    in_specs=[
        pl.BlockSpec((ROW_BLK, D_HID), lambda i: (i, 0)),
        pl.BlockSpec((KP, NC, ROW_BLK, 128), lambda i: (0, 0, i, 0)),
        _full((D_HID, D_HID)),
        _full((1, D_HID)),
        _full((D_HID, D_MLP)),
        _full((1, D_MLP)),
        _full((D_MLP, N_CLASS)),
        _full((1, N_CLASS)),
    ],
    out_specs=_full((N_NODES, N_CLASS)),
    out_shape=jax.ShapeDtypeStruct((N_NODES, N_CLASS), jnp.float32),
)


def kernel(nodes, edges, W_in, b_in, W_gcn, b_gcn, W1, b1, W2, b2):
    edges4 = edges.astype(jnp.int32).reshape(2, NC * NS, CHUNKS, EDGE_CHUNK)
    h0 = _tc_h0(nodes, W_in, b_in.reshape(1, D_HID))
    h0v = h0.reshape(KP * N_NODES, 128)
    partials = _sc_scatter(h0v, edges4)
    return _tc_dense(h0, partials,
                     W_gcn, b_gcn.reshape(1, D_HID),
                     W1, b1.reshape(1, 256),
                     W2, b2.reshape(1, N_CLASS))
